# Initial kernel scaffold; baseline (speedup 1.0000x reference)
import jax
import jax.numpy as jnp
from jax.experimental import pallas as pl


def kernel(s, kernel):
    raise NotImplementedError("write your pallas kernel here")


# speedup vs baseline: 2.6033x; 2.6033x over previous
import functools

import jax
import jax.numpy as jnp
from jax import lax
from jax.experimental import pallas as pl
from jax.experimental.pallas import tpu as pltpu
from jax.experimental.pallas import tpu_sc as plsc

L_SPINS = 20
BATCH = 16384
DELTA = 1e-15
LN2 = 0.6931471805599453
PI = 3.141592653589793

NC = 2
NS = 16
LANES = 16
NW = NC * NS
B_PER_W = BATCH // NW
CHUNK = 128
NCHUNK = B_PER_W // CHUNK


def _log_angle(k):
    a = jnp.abs(k + jnp.float32(DELTA))
    bits = lax.bitcast_convert_type(a, jnp.int32)
    e = ((bits >> 23) & 0xFF).astype(jnp.float32) - 127.0
    m = lax.bitcast_convert_type((bits & 0x7FFFFF) | 0x3F800000, jnp.float32)
    t = (m - 1.0) / (m + 1.0)
    t2 = t * t
    poly = t * (2.0 + t2 * (2.0 / 3.0 + t2 * (2.0 / 5.0 + t2 * (2.0 / 7.0 + t2 * (2.0 / 9.0)))))
    re = e * jnp.float32(LN2) + poly
    im = jnp.where(k < 0.0, jnp.float32(PI), jnp.float32(0.0))
    return re, im


@functools.partial(
    pl.kernel,
    mesh=plsc.VectorSubcoreMesh(core_axis_name="c", subcore_axis_name="s"),
    out_type=[
        jax.ShapeDtypeStruct((BATCH,), jnp.float32),
        jax.ShapeDtypeStruct((BATCH,), jnp.float32),
    ],
    scratch_types=[
        pltpu.VMEM((L_SPINS, B_PER_W), jnp.int32),
        pltpu.VMEM((NCHUNK, CHUNK), jnp.int32),
        pltpu.VMEM((B_PER_W,), jnp.float32),
        pltpu.VMEM((B_PER_W,), jnp.float32),
        pltpu.VMEM((B_PER_W,), jnp.float32),
        pltpu.SemaphoreType.DMA,
    ],
)
def _sc_lookup(s_hbm, table_hbm, re_hbm, im_hbm,
               s_v, idx_v, k_v, re_v, im_v, sem):
    wid = lax.axis_index("s") * NC + lax.axis_index("c")
    base = wid * B_PER_W
    pltpu.sync_copy(s_hbm.at[:, pl.ds(base, B_PER_W)], s_v)

    for j in range(B_PER_W // LANES):
        sl = pl.ds(j * LANES, LANES)
        acc = s_v[0, sl]
        for l in range(1, L_SPINS):
            acc = acc + acc + s_v[l, sl]
        idx_v[j // (CHUNK // LANES), pl.ds((j % (CHUNK // LANES)) * LANES, LANES)] = acc

    copies = [
        pltpu.async_copy(table_hbm.at[idx_v.at[c]],
                         k_v.at[pl.ds(c * CHUNK, CHUNK)], sem)
        for c in range(NCHUNK)
    ]
    for cp in copies:
        cp.wait()

    for j in range(B_PER_W // LANES):
        sl = pl.ds(j * LANES, LANES)
        re, im = _log_angle(k_v[sl])
        re_v[sl] = re
        im_v[sl] = im

    pltpu.sync_copy(re_v, re_hbm.at[pl.ds(base, B_PER_W)])
    pltpu.sync_copy(im_v, im_hbm.at[pl.ds(base, B_PER_W)])


def kernel(s, kernel):
    re, im = _sc_lookup(s, kernel)
    return lax.complex(re, im)


# speedup vs baseline: 2.6461x; 1.0164x over previous
import functools

import jax
import jax.numpy as jnp
from jax import lax
from jax.experimental import pallas as pl
from jax.experimental.pallas import tpu as pltpu
from jax.experimental.pallas import tpu_sc as plsc

L_SPINS = 20
BATCH = 16384
DELTA = 1e-15
LN2 = 0.6931471805599453
PI = 3.141592653589793

NC = 2
NS = 16
LANES = 16
NW = NC * NS
B_PER_W = BATCH // NW
CHUNK = 128
NCHUNK = B_PER_W // CHUNK


def _log_angle(k):
    a = jnp.abs(k + jnp.float32(DELTA))
    bits = lax.bitcast_convert_type(a, jnp.int32)
    e = ((bits >> 23) & 0xFF).astype(jnp.float32) - 127.0
    m = lax.bitcast_convert_type((bits & 0x7FFFFF) | 0x3F800000, jnp.float32)
    t = (m - 1.0) / (m + 1.0)
    t2 = t * t
    poly = t * (2.0 + t2 * (2.0 / 3.0 + t2 * (2.0 / 5.0 + t2 * (2.0 / 7.0 + t2 * (2.0 / 9.0)))))
    re = e * jnp.float32(LN2) + poly
    im = jnp.where(k < 0.0, jnp.float32(PI), jnp.float32(0.0))
    return re, im


@functools.partial(
    pl.kernel,
    mesh=plsc.VectorSubcoreMesh(core_axis_name="c", subcore_axis_name="s"),
    out_type=[
        jax.ShapeDtypeStruct((BATCH,), jnp.float32),
        jax.ShapeDtypeStruct((BATCH,), jnp.float32),
    ],
    scratch_types=[
        pltpu.VMEM((L_SPINS, B_PER_W), jnp.int32),
        pltpu.VMEM((NCHUNK, CHUNK), jnp.int32),
        pltpu.VMEM((B_PER_W,), jnp.float32),
        pltpu.VMEM((B_PER_W,), jnp.float32),
        pltpu.VMEM((B_PER_W,), jnp.float32),
        pltpu.SemaphoreType.DMA,
    ],
)
def _sc_lookup(s_hbm, table_hbm, re_hbm, im_hbm,
               s_v, idx_v, k_v, re_v, im_v, sem):
    wid = lax.axis_index("s") * NC + lax.axis_index("c")
    base = wid * B_PER_W
    pltpu.sync_copy(s_hbm.at[:, pl.ds(base, B_PER_W)], s_v)

    gathers = []
    for c in range(NCHUNK):
        def horner(j, carry, c=c):
            sl = pl.ds(c * CHUNK + j * LANES, LANES)
            acc = s_v[0, sl]
            for l in range(1, L_SPINS):
                acc = acc + acc + s_v[l, sl]
            idx_v[c, pl.ds(j * LANES, LANES)] = acc
            return carry
        lax.fori_loop(0, CHUNK // LANES, horner, 0, unroll=2)
        gathers.append(pltpu.async_copy(table_hbm.at[idx_v.at[c]],
                                        k_v.at[pl.ds(c * CHUNK, CHUNK)], sem))

    for c in range(NCHUNK):
        gathers[c].wait()

        def epilogue(j, carry, c=c):
            sl = pl.ds(c * CHUNK + j * LANES, LANES)
            re, im = _log_angle(k_v[sl])
            re_v[sl] = re
            im_v[sl] = im
            return carry
        lax.fori_loop(0, CHUNK // LANES, epilogue, 0, unroll=2)

    pltpu.sync_copy(re_v, re_hbm.at[pl.ds(base, B_PER_W)])
    pltpu.sync_copy(im_v, im_hbm.at[pl.ds(base, B_PER_W)])


def kernel(s, kernel):
    re, im = _sc_lookup(s, kernel)
    return lax.complex(re, im)


# speedup vs baseline: 2.7176x; 1.0270x over previous
import functools

import jax
import jax.numpy as jnp
from jax import lax
from jax.experimental import pallas as pl
from jax.experimental.pallas import tpu as pltpu
from jax.experimental.pallas import tpu_sc as plsc

L_SPINS = 20
BATCH = 16384
DELTA = 1e-15
LN2 = 0.6931471805599453
PI = 3.141592653589793

NC = 2
NS = 16
LANES = 16
NW = NC * NS
B_PER_W = BATCH // NW
CHUNK = 128
NCHUNK = B_PER_W // CHUNK


def _log_angle(k):
    a = jnp.abs(k + jnp.float32(DELTA))
    bits = lax.bitcast_convert_type(a, jnp.int32)
    e = ((bits >> 23) & 0xFF).astype(jnp.float32) - 127.0
    m = lax.bitcast_convert_type((bits & 0x7FFFFF) | 0x3F800000, jnp.float32)
    t = (m - 1.0) / (m + 1.0)
    t2 = t * t
    poly = t * (2.0 + t2 * (2.0 / 3.0 + t2 * (2.0 / 5.0 + t2 * (2.0 / 7.0 + t2 * (2.0 / 9.0)))))
    re = e * jnp.float32(LN2) + poly
    im = jnp.where(k < 0.0, jnp.float32(PI), jnp.float32(0.0))
    return re, im


@functools.partial(
    pl.kernel,
    mesh=plsc.VectorSubcoreMesh(core_axis_name="c", subcore_axis_name="s"),
    out_type=jax.ShapeDtypeStruct((2, BATCH), jnp.float32),
    scratch_types=[
        pltpu.VMEM((L_SPINS, B_PER_W), jnp.int32),
        pltpu.VMEM((NCHUNK, CHUNK), jnp.int32),
        pltpu.VMEM((B_PER_W,), jnp.float32),
        pltpu.VMEM((2, B_PER_W), jnp.float32),
        pltpu.SemaphoreType.DMA,
        pltpu.SemaphoreType.DMA,
    ],
)
def _sc_lookup(s_hbm, table_hbm, out_hbm,
               s_v, idx_v, k_v, ri_v, gsem, osem):
    wid = lax.axis_index("s") * NC + lax.axis_index("c")
    base = wid * B_PER_W

    s_loads = [
        pltpu.async_copy(s_hbm.at[:, pl.ds(base + c * CHUNK, CHUNK)],
                         s_v.at[:, pl.ds(c * CHUNK, CHUNK)], osem)
        for c in range(NCHUNK)
    ]

    gathers = []
    for c in range(NCHUNK):
        s_loads[c].wait()

        def horner(j, carry, c=c):
            sl = pl.ds(c * CHUNK + j * LANES, LANES)
            acc = s_v[0, sl]
            for l in range(1, L_SPINS):
                acc = acc + acc + s_v[l, sl]
            idx_v[c, pl.ds(j * LANES, LANES)] = acc
            return carry
        lax.fori_loop(0, CHUNK // LANES, horner, 0, unroll=2)
        gathers.append(pltpu.async_copy(table_hbm.at[idx_v.at[c]],
                                        k_v.at[pl.ds(c * CHUNK, CHUNK)], gsem))

    for c in range(NCHUNK):
        gathers[c].wait()

        def epilogue(j, carry, c=c):
            sl = pl.ds(c * CHUNK + j * LANES, LANES)
            re, im = _log_angle(k_v[sl])
            ri_v[0, sl] = re
            ri_v[1, sl] = im
            return carry
        lax.fori_loop(0, CHUNK // LANES, epilogue, 0, unroll=2)

    pltpu.async_copy(ri_v, out_hbm.at[:, pl.ds(base, B_PER_W)], osem).wait()


def kernel(s, kernel):
    ri = _sc_lookup(s, kernel)
    return lax.complex(ri[0], ri[1])


# speedup vs baseline: 2.8991x; 1.0668x over previous
import functools

import jax
import jax.numpy as jnp
from jax import lax
from jax.experimental import pallas as pl
from jax.experimental.pallas import tpu as pltpu
from jax.experimental.pallas import tpu_sc as plsc

L_SPINS = 20
BATCH = 16384
DELTA = 1e-15
LN2 = 0.6931471805599453
PI = 3.141592653589793

NC = 2
NS = 16
LANES = 16
NW = NC * NS
B_PER_W = BATCH // NW
CHUNK = 128
NCHUNK = B_PER_W // CHUNK


def _log_angle(k):
    a = jnp.abs(k + jnp.float32(DELTA))
    bits = lax.bitcast_convert_type(a, jnp.int32)
    e = ((bits >> 23) & 0xFF).astype(jnp.float32) - 127.0
    m = lax.bitcast_convert_type((bits & 0x7FFFFF) | 0x3F800000, jnp.float32)
    t = (m - 1.0) / (m + 1.0)
    t2 = t * t
    poly = t * (2.0 + t2 * (2.0 / 3.0 + t2 * (2.0 / 5.0 + t2 * (2.0 / 7.0 + t2 * (2.0 / 9.0)))))
    re = e * jnp.float32(LN2) + poly
    im = jnp.where(k < 0.0, jnp.float32(PI), jnp.float32(0.0))
    return re, im


def _idx_body(s_ref, idx_ref):
    w = jnp.int32(1) << (jnp.int32(L_SPINS - 1)
                         - lax.broadcasted_iota(jnp.int32, (L_SPINS, 1), 0))
    idx_ref[...] = jnp.sum(s_ref[...] * w, axis=0, keepdims=True)


_tc_indices = pl.pallas_call(
    _idx_body,
    out_shape=jax.ShapeDtypeStruct((1, BATCH), jnp.int32),
)


@functools.partial(
    pl.kernel,
    mesh=plsc.VectorSubcoreMesh(core_axis_name="c", subcore_axis_name="s"),
    out_type=jax.ShapeDtypeStruct((2, BATCH), jnp.float32),
    scratch_types=[
        pltpu.VMEM((NCHUNK, CHUNK), jnp.int32),
        pltpu.VMEM((B_PER_W,), jnp.float32),
        pltpu.VMEM((2, B_PER_W), jnp.float32),
        pltpu.SemaphoreType.DMA,
        pltpu.SemaphoreType.DMA,
    ],
)
def _sc_lookup(idx_hbm, table_hbm, out_hbm,
               idx_v, k_v, ri_v, gsem, osem):
    wid = lax.axis_index("s") * NC + lax.axis_index("c")
    base = wid * B_PER_W

    idx_loads = [
        pltpu.async_copy(idx_hbm.at[0, pl.ds(base + c * CHUNK, CHUNK)],
                         idx_v.at[c], osem)
        for c in range(NCHUNK)
    ]
    gathers = []
    for c in range(NCHUNK):
        idx_loads[c].wait()
        gathers.append(pltpu.async_copy(table_hbm.at[idx_v.at[c]],
                                        k_v.at[pl.ds(c * CHUNK, CHUNK)], gsem))

    for c in range(NCHUNK):
        gathers[c].wait()

        def epilogue(j, carry, c=c):
            sl = pl.ds(c * CHUNK + j * LANES, LANES)
            re, im = _log_angle(k_v[sl])
            ri_v[0, sl] = re
            ri_v[1, sl] = im
            return carry
        lax.fori_loop(0, CHUNK // LANES, epilogue, 0, unroll=2)

    pltpu.async_copy(ri_v, out_hbm.at[:, pl.ds(base, B_PER_W)], osem).wait()


def kernel(s, kernel):
    idx = _tc_indices(s)
    ri = _sc_lookup(idx, kernel)
    return lax.complex(ri[0], ri[1])


# speedup vs baseline: 2.9338x; 1.0120x over previous
import functools

import jax
import jax.numpy as jnp
from jax import lax
from jax.experimental import pallas as pl
from jax.experimental.pallas import tpu as pltpu
from jax.experimental.pallas import tpu_sc as plsc

L_SPINS = 20
BATCH = 16384
DELTA = 1e-15
LN2 = 0.6931471805599453
PI = 3.141592653589793

NC = 2
NS = 16
LANES = 16
NW = NC * NS
B_PER_W = BATCH // NW
CHUNK = 128
NCHUNK = B_PER_W // CHUNK


def _log_angle(k):
    a = jnp.abs(k + jnp.float32(DELTA))
    bits = lax.bitcast_convert_type(a, jnp.int32)
    e = ((bits >> 23) & 0xFF).astype(jnp.float32) - 127.0
    m = lax.bitcast_convert_type((bits & 0x7FFFFF) | 0x3F800000, jnp.float32)
    t = (m - 1.0) / (m + 1.0)
    t2 = t * t
    poly = t * (2.0 + t2 * (2.0 / 3.0 + t2 * (2.0 / 5.0 + t2 * (2.0 / 7.0 + t2 * (2.0 / 9.0)))))
    re = e * jnp.float32(LN2) + poly
    im = jnp.where(k < 0.0, jnp.float32(PI), jnp.float32(0.0))
    return re, im


def _idx_body(s_ref, idx_ref):
    w = jnp.int32(1) << (jnp.int32(L_SPINS - 1)
                         - lax.broadcasted_iota(jnp.int32, (L_SPINS, 1), 0))
    idx_ref[...] = jnp.sum(s_ref[...] * w, axis=0, keepdims=True)


_tc_indices = pl.pallas_call(
    _idx_body,
    grid=(4,),
    in_specs=[pl.BlockSpec((L_SPINS, BATCH // 4), lambda i: (0, i))],
    out_specs=pl.BlockSpec((1, BATCH // 4), lambda i: (0, i)),
    out_shape=jax.ShapeDtypeStruct((1, BATCH), jnp.int32),
)


@functools.partial(
    pl.kernel,
    mesh=plsc.VectorSubcoreMesh(core_axis_name="c", subcore_axis_name="s"),
    out_type=jax.ShapeDtypeStruct((2, BATCH), jnp.float32),
    scratch_types=[
        pltpu.VMEM((NCHUNK, CHUNK), jnp.int32),
        pltpu.VMEM((B_PER_W,), jnp.float32),
        pltpu.VMEM((2, B_PER_W), jnp.float32),
        pltpu.SemaphoreType.DMA,
        pltpu.SemaphoreType.DMA,
    ],
)
def _sc_lookup(idx_hbm, table_hbm, out_hbm,
               idx_v, k_v, ri_v, gsem, osem):
    wid = lax.axis_index("s") * NC + lax.axis_index("c")
    base = wid * B_PER_W

    idx_loads = [
        pltpu.async_copy(idx_hbm.at[0, pl.ds(base + c * CHUNK, CHUNK)],
                         idx_v.at[c], osem)
        for c in range(NCHUNK)
    ]
    gathers = []
    for c in range(NCHUNK):
        idx_loads[c].wait()
        gathers.append(pltpu.async_copy(table_hbm.at[idx_v.at[c]],
                                        k_v.at[pl.ds(c * CHUNK, CHUNK)], gsem))

    outs = []
    for c in range(NCHUNK):
        gathers[c].wait()

        def epilogue(j, carry, c=c):
            sl = pl.ds(c * CHUNK + j * LANES, LANES)
            re, im = _log_angle(k_v[sl])
            ri_v[0, sl] = re
            ri_v[1, sl] = im
            return carry
        lax.fori_loop(0, CHUNK // LANES, epilogue, 0)
        outs.append(pltpu.async_copy(
            ri_v.at[:, pl.ds(c * CHUNK, CHUNK)],
            out_hbm.at[:, pl.ds(base + c * CHUNK, CHUNK)], osem))
    for o in outs:
        o.wait()


def kernel(s, kernel):
    idx = _tc_indices(s)
    ri = _sc_lookup(idx, kernel)
    return lax.complex(ri[0], ri[1])


# speedup vs baseline: 2.9787x; 1.0153x over previous
import functools

import jax
import jax.numpy as jnp
from jax import lax
from jax.experimental import pallas as pl
from jax.experimental.pallas import tpu as pltpu
from jax.experimental.pallas import tpu_sc as plsc

L_SPINS = 20
BATCH = 16384
DELTA = 1e-15
LN2 = 0.6931471805599453
PI = 3.141592653589793

NC = 2
NS = 16
LANES = 16
NW = NC * NS
B_PER_W = BATCH // NW
CHUNK = 128
NCHUNK = B_PER_W // CHUNK


def _log_angle(k):
    a = jnp.abs(k + jnp.float32(DELTA))
    bits = lax.bitcast_convert_type(a, jnp.int32)
    e = ((bits >> 23) & 0xFF).astype(jnp.float32) - 127.0
    m = lax.bitcast_convert_type((bits & 0x7FFFFF) | 0x3F800000, jnp.float32)
    t = (m - 1.0) / (m + 1.0)
    t2 = t * t
    poly = t * (2.0 + t2 * (2.0 / 3.0 + t2 * (2.0 / 5.0 + t2 * (2.0 / 7.0 + t2 * (2.0 / 9.0)))))
    re = e * jnp.float32(LN2) + poly
    im = jnp.where(k < 0.0, jnp.float32(PI), jnp.float32(0.0))
    return re, im


def _idx_body(s_ref, idx_ref):
    w = jnp.int32(1) << (jnp.int32(L_SPINS - 1)
                         - lax.broadcasted_iota(jnp.int32, (L_SPINS, 1), 0))
    idx_ref[...] = jnp.sum(s_ref[...] * w, axis=0, keepdims=True)


_tc_indices = pl.pallas_call(
    _idx_body,
    grid=(2,),
    in_specs=[pl.BlockSpec((L_SPINS, BATCH // 2), lambda i: (0, i))],
    out_specs=pl.BlockSpec((1, BATCH // 2), lambda i: (0, i)),
    out_shape=jax.ShapeDtypeStruct((1, BATCH), jnp.int32),
)


@functools.partial(
    pl.kernel,
    mesh=plsc.VectorSubcoreMesh(core_axis_name="c", subcore_axis_name="s"),
    out_type=jax.ShapeDtypeStruct((2, BATCH), jnp.float32),
    scratch_types=[
        pltpu.VMEM((NCHUNK, CHUNK), jnp.int32),
        pltpu.VMEM((B_PER_W,), jnp.float32),
        pltpu.VMEM((2, B_PER_W), jnp.float32),
        pltpu.SemaphoreType.DMA,
        pltpu.SemaphoreType.DMA,
    ],
)
def _sc_lookup(idx_hbm, table_hbm, out_hbm,
               idx_v, k_v, ri_v, gsem, osem):
    wid = lax.axis_index("s") * NC + lax.axis_index("c")
    base = wid * B_PER_W

    idx_loads = [
        pltpu.async_copy(idx_hbm.at[0, pl.ds(base + c * CHUNK, CHUNK)],
                         idx_v.at[c], osem)
        for c in range(NCHUNK)
    ]
    gathers = []
    for c in range(NCHUNK):
        idx_loads[c].wait()
        gathers.append(pltpu.async_copy(table_hbm.at[idx_v.at[c]],
                                        k_v.at[pl.ds(c * CHUNK, CHUNK)], gsem))

    outs = []
    for c in range(NCHUNK):
        gathers[c].wait()

        def epilogue(j, carry, c=c):
            sl = pl.ds(c * CHUNK + j * LANES, LANES)
            re, im = _log_angle(k_v[sl])
            ri_v[0, sl] = re
            ri_v[1, sl] = im
            return carry
        lax.fori_loop(0, CHUNK // LANES, epilogue, 0)
        outs.append(pltpu.async_copy(
            ri_v.at[:, pl.ds(c * CHUNK, CHUNK)],
            out_hbm.at[:, pl.ds(base + c * CHUNK, CHUNK)], osem))
    for o in outs:
        o.wait()


def kernel(s, kernel):
    idx = _tc_indices(s)
    ri = _sc_lookup(idx, kernel)
    return lax.complex(ri[0], ri[1])
